# baseline (speedup 1.0000x reference)
import functools

import jax
import jax.numpy as jnp
from jax import lax
from jax.experimental import pallas as pl
from jax.experimental.pallas import tpu as pltpu
from jax.experimental.pallas import tpu_sc as plsc

NC = 2
NS = 16
LANES = 16
C = 125
NSLOT = 8
LOOK = 3


def _round_up(v, m):
    return (v + m - 1) // m * m


def _make_sc_agg(n, e, dh, with_count):
    ew = e // NS
    n_chunks = ew // C
    ZR = 128
    np_ = _round_up(n, NS * ZR)
    rpt = np_ // NS
    nz = rpt // ZR
    assert ew % C == 0 and n_chunks % NSLOT == 0 and 0 < LOOK < NSLOT
    assert n_chunks % 2 == 0

    mesh = plsc.VectorSubcoreMesh(
        core_axis_name="c", subcore_axis_name="s",
        num_cores=NC, num_subcores=NS)

    outs = [jax.ShapeDtypeStruct((np_, dh), jnp.bfloat16),
            jax.ShapeDtypeStruct((np_, dh), jnp.bfloat16)]
    scratch = [
        pltpu.VMEM((n_chunks, C), jnp.int32),
        pltpu.VMEM((n_chunks, C), jnp.int32),
        pltpu.VMEM((NSLOT, C, dh), jnp.bfloat16),
        pltpu.VMEM((ZR, dh), jnp.bfloat16),
        pltpu.VMEM_SHARED((np_, dh), jnp.bfloat16),
        [pltpu.SemaphoreType.DMA] * NSLOT,
        [pltpu.SemaphoreType.DMA] * NSLOT,
    ]
    if with_count:
        outs += [jax.ShapeDtypeStruct((np_, 16), jnp.float32),
                 jax.ShapeDtypeStruct((np_, 16), jnp.float32)]
        scratch += [
            pltpu.VMEM((C, 16), jnp.float32),
            pltpu.VMEM((ZR, 16), jnp.float32),
            pltpu.VMEM_SHARED((np_, 16), jnp.float32),
            pltpu.SemaphoreType.DMA,
        ]

    @functools.partial(
        pl.kernel, out_type=outs, mesh=mesh, scratch_types=scratch,
        compiler_params=pltpu.CompilerParams(use_tc_tiling_on_sc=False))
    def sc_agg(y0, y1, src_hbm, dst_hbm, *refs):
        if with_count:
            (o0, o1, c0, c1, src_v, dst_v, rows_v, zb, acc,
             gsems, ssems, ones_v, zc, cacc, csem) = refs
        else:
            (o0, o1, src_v, dst_v, rows_v, zb, acc, gsems, ssems) = refs
        ci = lax.axis_index("c")
        si = lax.axis_index("s")

        pltpu.sync_copy(src_hbm.at[si], src_v)
        pltpu.sync_copy(dst_hbm.at[si], dst_v)

        zrow16 = jnp.zeros((2 * LANES,), jnp.bfloat16)

        @pl.loop(0, ZR)
        def _(r):
            for j in range(dh // (2 * LANES)):
                zb[r, pl.ds(j * 2 * LANES, 2 * LANES)] = zrow16

        if with_count:
            zrow = jnp.zeros((LANES,), jnp.float32)
            onerow = jnp.ones((LANES,), jnp.float32)

            @pl.loop(0, ZR)
            def _(r):
                zc[r, :] = zrow

            @pl.loop(0, C)
            def _(r):
                ones_v[r, :] = onerow

        row0 = si * rpt
        for j in range(nz):
            pltpu.sync_copy(zb, acc.at[pl.ds(row0 + j * ZR, ZR)])
        if with_count:
            for j in range(nz):
                pltpu.sync_copy(zc, cacc.at[pl.ds(row0 + j * ZR, ZR)])
        plsc.subcore_barrier()

        def run(y_ref):
            def issue_gather(i, b):
                pltpu.async_copy(y_ref.at[src_v.at[i]], rows_v.at[b],
                                 gsems[b])

            def wait_gather(b):
                pltpu.make_async_copy(y_ref.at[src_v.at[0]], rows_v.at[b],
                                      gsems[b]).wait()

            def wait_scatter(b):
                pltpu.make_async_copy(rows_v.at[b], acc.at[dst_v.at[0]],
                                      ssems[b]).wait()

            for b in range(LOOK):
                issue_gather(b, b)

            @pl.loop(0, n_chunks // NSLOT)
            def _(g):
                for b in range(NSLOT):
                    i = g * NSLOT + b
                    wait_gather(b)
                    pltpu.async_copy(rows_v.at[b], acc.at[dst_v.at[i]],
                                     ssems[b], add=True)
                    if with_count:
                        @pl.when(ci == lax.rem(i, 2))
                        def _():
                            pltpu.async_copy(ones_v, cacc.at[dst_v.at[i]],
                                             csem, add=True)
                    sj = (b + LOOK) % NSLOT
                    if b < NSLOT - LOOK:
                        @pl.when(g > 0)
                        def _():
                            wait_scatter(sj)
                        issue_gather(i + LOOK, sj)
                    else:
                        wait_scatter(sj)

                        @pl.when(i + LOOK < n_chunks)
                        def _():
                            issue_gather(i + LOOK, sj)

            for b in range(LOOK, NSLOT):
                wait_scatter(b)

            if with_count:
                @pl.loop(0, n_chunks // 2)
                def _(i):
                    pltpu.make_async_copy(ones_v, cacc.at[dst_v.at[0]],
                                          csem).wait()

        @pl.when(ci == 0)
        def _():
            run(y0)

        @pl.when(ci == 1)
        def _():
            run(y1)

        plsc.subcore_barrier()

        def copy_out(o_ref, a_ref, buf):
            for j in range(nz):
                r = row0 + j * ZR
                pltpu.sync_copy(a_ref.at[pl.ds(r, ZR)], buf)
                pltpu.sync_copy(buf, o_ref.at[pl.ds(r, ZR)])

        @pl.when(ci == 0)
        def _():
            copy_out(o0, acc, zb)
            if with_count:
                copy_out(c0, cacc, zc)

        @pl.when(ci == 1)
        def _():
            copy_out(o1, acc, zb)
            if with_count:
                copy_out(c1, cacc, zc)

    return sc_agg


def _dotT(a, w):
    return lax.dot_general(a, w, (((1,), (1,)), ((), ())),
                           preferred_element_type=jnp.float32)


def _tc_pre(x, wl):
    n, din = x.shape
    d = wl.shape[0]
    dh = d // 2
    B = 1000

    def body(x_ref, w_ref, o0_ref, o1_ref):
        y = _dotT(x_ref[...], w_ref[...]).astype(jnp.bfloat16)
        o0_ref[...] = y[:, :dh]
        o1_ref[...] = y[:, dh:]

    out = jax.ShapeDtypeStruct((n, dh), jnp.bfloat16)
    return pl.pallas_call(
        body,
        grid=(n // B,),
        in_specs=[pl.BlockSpec((B, din), lambda i: (i, 0)),
                  pl.BlockSpec((d, din), lambda i: (0, 0))],
        out_specs=[pl.BlockSpec((B, dh), lambda i: (i, 0)),
                   pl.BlockSpec((B, dh), lambda i: (i, 0))],
        out_shape=[out, out],
    )(x, wl)


def _tc_mid(s0, s1, c0, c1, h_prev, wr, b, wl_next):
    n, din = h_prev.shape
    d = wr.shape[0]
    dh = s0.shape[1]
    dn = wl_next.shape[0]
    dhn = dn // 2
    B = 1000

    def body(s0_ref, s1_ref, c0_ref, c1_ref, h_ref, wr_ref, b_ref, wl_ref,
             h_out, y0_out, y1_out):
        s = jnp.concatenate([s0_ref[...], s1_ref[...]],
                            axis=1).astype(jnp.float32)
        cnt = c0_ref[:, 0:1] + c1_ref[:, 0:1]
        inv = 1.0 / jnp.maximum(cnt, 1.0)
        h = s * inv + _dotT(h_ref[...], wr_ref[...]) + b_ref[...]
        h = jnp.maximum(h, 0.0)
        h_out[...] = h
        y = _dotT(h, wl_ref[...]).astype(jnp.bfloat16)
        y0_out[...] = y[:, :dhn]
        y1_out[...] = y[:, dhn:]

    outs = [jax.ShapeDtypeStruct((n, d), jnp.float32),
            jax.ShapeDtypeStruct((n, dhn), jnp.bfloat16),
            jax.ShapeDtypeStruct((n, dhn), jnp.bfloat16)]
    return pl.pallas_call(
        body,
        grid=(n // B,),
        in_specs=[pl.BlockSpec((B, dh), lambda i: (i, 0)),
                  pl.BlockSpec((B, dh), lambda i: (i, 0)),
                  pl.BlockSpec((B, 16), lambda i: (i, 0)),
                  pl.BlockSpec((B, 16), lambda i: (i, 0)),
                  pl.BlockSpec((B, din), lambda i: (i, 0)),
                  pl.BlockSpec((d, din), lambda i: (0, 0)),
                  pl.BlockSpec((1, d), lambda i: (0, 0)),
                  pl.BlockSpec((dn, d), lambda i: (0, 0))],
        out_specs=[pl.BlockSpec((B, d), lambda i: (i, 0)),
                   pl.BlockSpec((B, dhn), lambda i: (i, 0)),
                   pl.BlockSpec((B, dhn), lambda i: (i, 0))],
        out_shape=outs,
    )(s0, s1, c0, c1, h_prev, wr, b.reshape(1, d), wl_next)


def _tc_final(s0, s1, c0, c1, h_prev, wr, b):
    n, din = h_prev.shape
    d = wr.shape[0]
    dh = s0.shape[1]
    B = 1000

    def body(s0_ref, s1_ref, c0_ref, c1_ref, h_ref, wr_ref, b_ref, o_ref):
        s = jnp.concatenate([s0_ref[...], s1_ref[...]],
                            axis=1).astype(jnp.float32)
        cnt = c0_ref[:, 0:1] + c1_ref[:, 0:1]
        inv = 1.0 / jnp.maximum(cnt, 1.0)
        h = s * inv + _dotT(h_ref[...], wr_ref[...]) + b_ref[...]
        h = jnp.maximum(h, 0.0)
        m = jnp.max(h, axis=1, keepdims=True)
        lse = jnp.log(jnp.sum(jnp.exp(h - m), axis=1, keepdims=True))
        o_ref[...] = h - m - lse

    return pl.pallas_call(
        body,
        grid=(n // B,),
        in_specs=[pl.BlockSpec((B, dh), lambda i: (i, 0)),
                  pl.BlockSpec((B, dh), lambda i: (i, 0)),
                  pl.BlockSpec((B, 16), lambda i: (i, 0)),
                  pl.BlockSpec((B, 16), lambda i: (i, 0)),
                  pl.BlockSpec((B, din), lambda i: (i, 0)),
                  pl.BlockSpec((d, din), lambda i: (0, 0)),
                  pl.BlockSpec((1, d), lambda i: (0, 0))],
        out_specs=pl.BlockSpec((B, d), lambda i: (i, 0)),
        out_shape=jax.ShapeDtypeStruct((n, d), jnp.float32),
    )(s0, s1, c0, c1, h_prev, wr, b.reshape(1, d))


def kernel(x, edge_index, Wl1, bl1, Wr1, Wl2, bl2, Wr2, Wl3, bl3, Wr3):
    n = x.shape[0]
    e = edge_index.shape[1]
    src3 = edge_index[0].reshape(NS, (e // NS) // C, C)
    dst3 = edge_index[1].reshape(NS, (e // NS) // C, C)

    y10, y11 = _tc_pre(x, Wl1)
    s10, s11, c0, c1 = _make_sc_agg(n, e, Wl1.shape[0] // 2, True)(
        y10, y11, src3, dst3)
    h1, y20, y21 = _tc_mid(s10, s11, c0, c1, x, Wr1, bl1, Wl2)
    s20, s21 = _make_sc_agg(n, e, Wl2.shape[0] // 2, False)(
        y20, y21, src3, dst3)
    h2, y30, y31 = _tc_mid(s20, s21, c0, c1, h1, Wr2, bl2, Wl3)
    s30, s31 = _make_sc_agg(n, e, Wl3.shape[0] // 2, False)(
        y30, y31, src3, dst3)
    return _tc_final(s30, s31, c0, c1, h2, Wr3, bl3)


# speedup vs baseline: 1.0479x; 1.0479x over previous
import functools

import jax
import jax.numpy as jnp
from jax import lax
from jax.experimental import pallas as pl
from jax.experimental.pallas import tpu as pltpu
from jax.experimental.pallas import tpu_sc as plsc

NC = 2
NS = 16
LANES = 16
C = 80
NSLOT = 10
LOOK = 5


def _round_up(v, m):
    return (v + m - 1) // m * m


def _make_sc_agg(n, e, dh, with_count):
    ew = e // NS
    n_chunks = ew // C
    ZR = 128
    np_ = _round_up(n, NS * ZR)
    rpt = np_ // NS
    nz = rpt // ZR
    assert ew % C == 0 and n_chunks % NSLOT == 0 and 0 < LOOK < NSLOT
    assert n_chunks % 2 == 0

    mesh = plsc.VectorSubcoreMesh(
        core_axis_name="c", subcore_axis_name="s",
        num_cores=NC, num_subcores=NS)

    outs = [jax.ShapeDtypeStruct((np_, dh), jnp.bfloat16),
            jax.ShapeDtypeStruct((np_, dh), jnp.bfloat16)]
    scratch = [
        pltpu.VMEM((n_chunks, C), jnp.int32),
        pltpu.VMEM((n_chunks, C), jnp.int32),
        pltpu.VMEM((NSLOT, C, dh), jnp.bfloat16),
        pltpu.VMEM((ZR, dh), jnp.bfloat16),
        pltpu.VMEM_SHARED((np_, dh), jnp.bfloat16),
        [pltpu.SemaphoreType.DMA] * NSLOT,
        [pltpu.SemaphoreType.DMA] * NSLOT,
    ]
    if with_count:
        outs += [jax.ShapeDtypeStruct((np_, 16), jnp.float32),
                 jax.ShapeDtypeStruct((np_, 16), jnp.float32)]
        scratch += [
            pltpu.VMEM((C, 16), jnp.float32),
            pltpu.VMEM((ZR, 16), jnp.float32),
            pltpu.VMEM_SHARED((np_, 16), jnp.float32),
            pltpu.SemaphoreType.DMA,
        ]

    @functools.partial(
        pl.kernel, out_type=outs, mesh=mesh, scratch_types=scratch,
        compiler_params=pltpu.CompilerParams(use_tc_tiling_on_sc=False))
    def sc_agg(y0, y1, src_hbm, dst_hbm, *refs):
        if with_count:
            (o0, o1, c0, c1, src_v, dst_v, rows_v, zb, acc,
             gsems, ssems, ones_v, zc, cacc, csem) = refs
        else:
            (o0, o1, src_v, dst_v, rows_v, zb, acc, gsems, ssems) = refs
        ci = lax.axis_index("c")
        si = lax.axis_index("s")

        pltpu.sync_copy(src_hbm.at[si], src_v)
        pltpu.sync_copy(dst_hbm.at[si], dst_v)

        zrow16 = jnp.zeros((2 * LANES,), jnp.bfloat16)

        @pl.loop(0, ZR)
        def _(r):
            for j in range(dh // (2 * LANES)):
                zb[r, pl.ds(j * 2 * LANES, 2 * LANES)] = zrow16

        if with_count:
            zrow = jnp.zeros((LANES,), jnp.float32)
            onerow = jnp.ones((LANES,), jnp.float32)

            @pl.loop(0, ZR)
            def _(r):
                zc[r, :] = zrow

            @pl.loop(0, C)
            def _(r):
                ones_v[r, :] = onerow

        row0 = si * rpt
        for j in range(nz):
            pltpu.sync_copy(zb, acc.at[pl.ds(row0 + j * ZR, ZR)])
        if with_count:
            for j in range(nz):
                pltpu.sync_copy(zc, cacc.at[pl.ds(row0 + j * ZR, ZR)])
        plsc.subcore_barrier()

        def run(y_ref):
            def issue_gather(i, b):
                pltpu.async_copy(y_ref.at[src_v.at[i]], rows_v.at[b],
                                 gsems[b])

            def wait_gather(b):
                pltpu.make_async_copy(y_ref.at[src_v.at[0]], rows_v.at[b],
                                      gsems[b]).wait()

            def wait_scatter(b):
                pltpu.make_async_copy(rows_v.at[b], acc.at[dst_v.at[0]],
                                      ssems[b]).wait()

            for b in range(LOOK):
                issue_gather(b, b)

            @pl.loop(0, n_chunks // NSLOT)
            def _(g):
                for b in range(NSLOT):
                    i = g * NSLOT + b
                    wait_gather(b)
                    pltpu.async_copy(rows_v.at[b], acc.at[dst_v.at[i]],
                                     ssems[b], add=True)
                    if with_count:
                        @pl.when(ci == lax.rem(i, 2))
                        def _():
                            pltpu.async_copy(ones_v, cacc.at[dst_v.at[i]],
                                             csem, add=True)
                    sj = (b + LOOK) % NSLOT
                    if b < NSLOT - LOOK:
                        @pl.when(g > 0)
                        def _():
                            wait_scatter(sj)
                        issue_gather(i + LOOK, sj)
                    else:
                        wait_scatter(sj)

                        @pl.when(i + LOOK < n_chunks)
                        def _():
                            issue_gather(i + LOOK, sj)

            for b in range(LOOK, NSLOT):
                wait_scatter(b)

            if with_count:
                @pl.loop(0, n_chunks // 2)
                def _(i):
                    pltpu.make_async_copy(ones_v, cacc.at[dst_v.at[0]],
                                          csem).wait()

        @pl.when(ci == 0)
        def _():
            run(y0)

        @pl.when(ci == 1)
        def _():
            run(y1)

        plsc.subcore_barrier()

        def copy_out(o_ref, a_ref, buf):
            for j in range(nz):
                r = row0 + j * ZR
                pltpu.sync_copy(a_ref.at[pl.ds(r, ZR)], buf)
                pltpu.sync_copy(buf, o_ref.at[pl.ds(r, ZR)])

        @pl.when(ci == 0)
        def _():
            copy_out(o0, acc, zb)
            if with_count:
                copy_out(c0, cacc, zc)

        @pl.when(ci == 1)
        def _():
            copy_out(o1, acc, zb)
            if with_count:
                copy_out(c1, cacc, zc)

    return sc_agg


def _dotT(a, w):
    return lax.dot_general(a, w, (((1,), (1,)), ((), ())),
                           preferred_element_type=jnp.float32)


def _tc_pre(x, wl):
    n, din = x.shape
    d = wl.shape[0]
    dh = d // 2
    B = 1000

    def body(x_ref, w_ref, o0_ref, o1_ref):
        y = _dotT(x_ref[...], w_ref[...]).astype(jnp.bfloat16)
        o0_ref[...] = y[:, :dh]
        o1_ref[...] = y[:, dh:]

    out = jax.ShapeDtypeStruct((n, dh), jnp.bfloat16)
    return pl.pallas_call(
        body,
        grid=(n // B,),
        in_specs=[pl.BlockSpec((B, din), lambda i: (i, 0)),
                  pl.BlockSpec((d, din), lambda i: (0, 0))],
        out_specs=[pl.BlockSpec((B, dh), lambda i: (i, 0)),
                   pl.BlockSpec((B, dh), lambda i: (i, 0))],
        out_shape=[out, out],
    )(x, wl)


def _tc_mid(s0, s1, c0, c1, h_prev, wr, b, wl_next):
    n, din = h_prev.shape
    d = wr.shape[0]
    dh = s0.shape[1]
    dn = wl_next.shape[0]
    dhn = dn // 2
    B = 1000

    def body(s0_ref, s1_ref, c0_ref, c1_ref, h_ref, wr_ref, b_ref, wl_ref,
             h_out, y0_out, y1_out):
        s = jnp.concatenate([s0_ref[...], s1_ref[...]],
                            axis=1).astype(jnp.float32)
        cnt = c0_ref[:, 0:1] + c1_ref[:, 0:1]
        inv = 1.0 / jnp.maximum(cnt, 1.0)
        h = s * inv + _dotT(h_ref[...], wr_ref[...]) + b_ref[...]
        h = jnp.maximum(h, 0.0)
        h_out[...] = h
        y = _dotT(h, wl_ref[...]).astype(jnp.bfloat16)
        y0_out[...] = y[:, :dhn]
        y1_out[...] = y[:, dhn:]

    outs = [jax.ShapeDtypeStruct((n, d), jnp.float32),
            jax.ShapeDtypeStruct((n, dhn), jnp.bfloat16),
            jax.ShapeDtypeStruct((n, dhn), jnp.bfloat16)]
    return pl.pallas_call(
        body,
        grid=(n // B,),
        in_specs=[pl.BlockSpec((B, dh), lambda i: (i, 0)),
                  pl.BlockSpec((B, dh), lambda i: (i, 0)),
                  pl.BlockSpec((B, 16), lambda i: (i, 0)),
                  pl.BlockSpec((B, 16), lambda i: (i, 0)),
                  pl.BlockSpec((B, din), lambda i: (i, 0)),
                  pl.BlockSpec((d, din), lambda i: (0, 0)),
                  pl.BlockSpec((1, d), lambda i: (0, 0)),
                  pl.BlockSpec((dn, d), lambda i: (0, 0))],
        out_specs=[pl.BlockSpec((B, d), lambda i: (i, 0)),
                   pl.BlockSpec((B, dhn), lambda i: (i, 0)),
                   pl.BlockSpec((B, dhn), lambda i: (i, 0))],
        out_shape=outs,
    )(s0, s1, c0, c1, h_prev, wr, b.reshape(1, d), wl_next)


def _tc_final(s0, s1, c0, c1, h_prev, wr, b):
    n, din = h_prev.shape
    d = wr.shape[0]
    dh = s0.shape[1]
    B = 1000

    def body(s0_ref, s1_ref, c0_ref, c1_ref, h_ref, wr_ref, b_ref, o_ref):
        s = jnp.concatenate([s0_ref[...], s1_ref[...]],
                            axis=1).astype(jnp.float32)
        cnt = c0_ref[:, 0:1] + c1_ref[:, 0:1]
        inv = 1.0 / jnp.maximum(cnt, 1.0)
        h = s * inv + _dotT(h_ref[...], wr_ref[...]) + b_ref[...]
        h = jnp.maximum(h, 0.0)
        m = jnp.max(h, axis=1, keepdims=True)
        lse = jnp.log(jnp.sum(jnp.exp(h - m), axis=1, keepdims=True))
        o_ref[...] = h - m - lse

    return pl.pallas_call(
        body,
        grid=(n // B,),
        in_specs=[pl.BlockSpec((B, dh), lambda i: (i, 0)),
                  pl.BlockSpec((B, dh), lambda i: (i, 0)),
                  pl.BlockSpec((B, 16), lambda i: (i, 0)),
                  pl.BlockSpec((B, 16), lambda i: (i, 0)),
                  pl.BlockSpec((B, din), lambda i: (i, 0)),
                  pl.BlockSpec((d, din), lambda i: (0, 0)),
                  pl.BlockSpec((1, d), lambda i: (0, 0))],
        out_specs=pl.BlockSpec((B, d), lambda i: (i, 0)),
        out_shape=jax.ShapeDtypeStruct((n, d), jnp.float32),
    )(s0, s1, c0, c1, h_prev, wr, b.reshape(1, d))


def kernel(x, edge_index, Wl1, bl1, Wr1, Wl2, bl2, Wr2, Wl3, bl3, Wr3):
    n = x.shape[0]
    e = edge_index.shape[1]
    src3 = edge_index[0].reshape(NS, (e // NS) // C, C)
    dst3 = edge_index[1].reshape(NS, (e // NS) // C, C)

    y10, y11 = _tc_pre(x, Wl1)
    s10, s11, c0, c1 = _make_sc_agg(n, e, Wl1.shape[0] // 2, True)(
        y10, y11, src3, dst3)
    h1, y20, y21 = _tc_mid(s10, s11, c0, c1, x, Wr1, bl1, Wl2)
    s20, s21 = _make_sc_agg(n, e, Wl2.shape[0] // 2, False)(
        y20, y21, src3, dst3)
    h2, y30, y31 = _tc_mid(s20, s21, c0, c1, h1, Wr2, bl2, Wl3)
    s30, s31 = _make_sc_agg(n, e, Wl3.shape[0] // 2, False)(
        y30, y31, src3, dst3)
    return _tc_final(s30, s31, c0, c1, h2, Wr3, bl3)


# speedup vs baseline: 1.0838x; 1.0343x over previous
import functools

import jax
import jax.numpy as jnp
from jax import lax
from jax.experimental import pallas as pl
from jax.experimental.pallas import tpu as pltpu
from jax.experimental.pallas import tpu_sc as plsc

NC = 2
NS = 16
LANES = 16
C = 80
NSLOT = 10
LOOK = 5


def _round_up(v, m):
    return (v + m - 1) // m * m


def _make_sc_agg(n, e, dh, with_count):
    ew = e // NS
    n_chunks = ew // C
    ZR = 128
    np_ = _round_up(n, NS * ZR)
    rpt = np_ // NS
    nz = rpt // ZR
    assert ew % C == 0 and n_chunks % NSLOT == 0 and 0 < LOOK < NSLOT
    assert n_chunks % 2 == 0

    mesh = plsc.VectorSubcoreMesh(
        core_axis_name="c", subcore_axis_name="s",
        num_cores=NC, num_subcores=NS)

    outs = [jax.ShapeDtypeStruct((np_, dh), jnp.bfloat16),
            jax.ShapeDtypeStruct((np_, dh), jnp.bfloat16)]
    scratch = [
        pltpu.VMEM((n_chunks, C), jnp.int32),
        pltpu.VMEM((n_chunks, C), jnp.int32),
        pltpu.VMEM((NSLOT, C, dh), jnp.bfloat16),
        pltpu.VMEM((ZR, dh), jnp.bfloat16),
        pltpu.VMEM_SHARED((np_, dh), jnp.bfloat16),
        [pltpu.SemaphoreType.DMA] * NSLOT,
        [pltpu.SemaphoreType.DMA] * NSLOT,
    ]
    if with_count:
        outs += [jax.ShapeDtypeStruct((np_, 16), jnp.float32),
                 jax.ShapeDtypeStruct((np_, 16), jnp.float32)]
        scratch += [
            pltpu.VMEM((C, 16), jnp.float32),
            pltpu.VMEM((ZR, 16), jnp.float32),
            pltpu.VMEM_SHARED((np_, 16), jnp.float32),
            pltpu.SemaphoreType.DMA,
        ]

    @functools.partial(
        pl.kernel, out_type=outs, mesh=mesh, scratch_types=scratch,
        compiler_params=pltpu.CompilerParams(use_tc_tiling_on_sc=False))
    def sc_agg(y0, y1, src_hbm, dst_hbm, *refs):
        if with_count:
            (o0, o1, c0, c1, src_v, dst_v, rows_v, zb, acc,
             gsems, ssems, ones_v, zc, cacc, csem) = refs
        else:
            (o0, o1, src_v, dst_v, rows_v, zb, acc, gsems, ssems) = refs
        ci = lax.axis_index("c")
        si = lax.axis_index("s")

        pltpu.sync_copy(src_hbm.at[si], src_v)
        pltpu.sync_copy(dst_hbm.at[si], dst_v)

        zrow16 = jnp.zeros((2 * LANES,), jnp.bfloat16)

        @pl.loop(0, ZR)
        def _(r):
            for j in range(dh // (2 * LANES)):
                zb[r, pl.ds(j * 2 * LANES, 2 * LANES)] = zrow16

        if with_count:
            zrow = jnp.zeros((LANES,), jnp.float32)
            onerow = jnp.ones((LANES,), jnp.float32)

            @pl.loop(0, ZR)
            def _(r):
                zc[r, :] = zrow

            @pl.loop(0, C)
            def _(r):
                ones_v[r, :] = onerow

        row0 = si * rpt
        for j in range(nz):
            pltpu.sync_copy(zb, acc.at[pl.ds(row0 + j * ZR, ZR)])
        if with_count:
            for j in range(nz):
                pltpu.sync_copy(zc, cacc.at[pl.ds(row0 + j * ZR, ZR)])
        plsc.subcore_barrier()

        def run(y_ref):
            def issue_gather(i, b):
                pltpu.async_copy(y_ref.at[src_v.at[i]], rows_v.at[b],
                                 gsems[b])

            def wait_gather(b):
                pltpu.make_async_copy(y_ref.at[src_v.at[0]], rows_v.at[b],
                                      gsems[b]).wait()

            def wait_scatter(b):
                pltpu.make_async_copy(rows_v.at[b], acc.at[dst_v.at[0]],
                                      ssems[b]).wait()

            for b in range(LOOK):
                issue_gather(b, b)

            @pl.loop(0, n_chunks // NSLOT)
            def _(g):
                for b in range(NSLOT):
                    i = g * NSLOT + b
                    wait_gather(b)
                    pltpu.async_copy(rows_v.at[b], acc.at[dst_v.at[i]],
                                     ssems[b], add=True)
                    if with_count:
                        @pl.when(ci == lax.rem(i, 2))
                        def _():
                            pltpu.async_copy(ones_v, cacc.at[dst_v.at[i]],
                                             csem, add=True)
                    sj = (b + LOOK) % NSLOT
                    if b < NSLOT - LOOK:
                        @pl.when(g > 0)
                        def _():
                            wait_scatter(sj)
                        issue_gather(i + LOOK, sj)
                    else:
                        wait_scatter(sj)

                        @pl.when(i + LOOK < n_chunks)
                        def _():
                            issue_gather(i + LOOK, sj)

            for b in range(LOOK, NSLOT):
                wait_scatter(b)

            if with_count:
                @pl.loop(0, n_chunks // 2)
                def _(i):
                    pltpu.make_async_copy(ones_v, cacc.at[dst_v.at[0]],
                                          csem).wait()

        @pl.when(ci == 0)
        def _():
            run(y0)

        @pl.when(ci == 1)
        def _():
            run(y1)

        plsc.subcore_barrier()

        def copy_out(o_ref, a_ref, buf):
            for j in range(nz):
                r = row0 + j * ZR
                pltpu.sync_copy(a_ref.at[pl.ds(r, ZR)], buf)
                pltpu.sync_copy(buf, o_ref.at[pl.ds(r, ZR)])

        @pl.when(ci == 0)
        def _():
            copy_out(o0, acc, zb)
            if with_count:
                copy_out(c0, cacc, zc)

        @pl.when(ci == 1)
        def _():
            copy_out(o1, acc, zb)
            if with_count:
                copy_out(c1, cacc, zc)

    return sc_agg


def _dotT(a, w):
    return lax.dot_general(a, w, (((1,), (1,)), ((), ())),
                           preferred_element_type=jnp.float32)


def _tc_pre(x, wl):
    n, din = x.shape
    d = wl.shape[0]
    dh = d // 2
    B = 2000

    def body(x_ref, w_ref, o0_ref, o1_ref):
        y = _dotT(x_ref[...], w_ref[...]).astype(jnp.bfloat16)
        o0_ref[...] = y[:, :dh]
        o1_ref[...] = y[:, dh:]

    out = jax.ShapeDtypeStruct((n, dh), jnp.bfloat16)
    return pl.pallas_call(
        body,
        grid=(n // B,),
        in_specs=[pl.BlockSpec((B, din), lambda i: (i, 0)),
                  pl.BlockSpec((d, din), lambda i: (0, 0))],
        out_specs=[pl.BlockSpec((B, dh), lambda i: (i, 0)),
                   pl.BlockSpec((B, dh), lambda i: (i, 0))],
        out_shape=[out, out],
    )(x, wl)


def _tc_mid(s0, s1, c0, c1, h_prev, wr, b, wl_next):
    n, din = h_prev.shape
    d = wr.shape[0]
    dh = s0.shape[1]
    dn = wl_next.shape[0]
    dhn = dn // 2
    B = 2000

    def body(s0_ref, s1_ref, c0_ref, c1_ref, h_ref, wr_ref, b_ref, wl_ref,
             h_out, y0_out, y1_out):
        s = jnp.concatenate([s0_ref[...], s1_ref[...]],
                            axis=1).astype(jnp.float32)
        cnt = c0_ref[:, 0:1] + c1_ref[:, 0:1]
        inv = 1.0 / jnp.maximum(cnt, 1.0)
        h = s * inv + _dotT(h_ref[...], wr_ref[...]) + b_ref[...]
        h = jnp.maximum(h, 0.0)
        h_out[...] = h
        y = _dotT(h, wl_ref[...]).astype(jnp.bfloat16)
        y0_out[...] = y[:, :dhn]
        y1_out[...] = y[:, dhn:]

    outs = [jax.ShapeDtypeStruct((n, d), jnp.float32),
            jax.ShapeDtypeStruct((n, dhn), jnp.bfloat16),
            jax.ShapeDtypeStruct((n, dhn), jnp.bfloat16)]
    return pl.pallas_call(
        body,
        grid=(n // B,),
        in_specs=[pl.BlockSpec((B, dh), lambda i: (i, 0)),
                  pl.BlockSpec((B, dh), lambda i: (i, 0)),
                  pl.BlockSpec((B, 16), lambda i: (i, 0)),
                  pl.BlockSpec((B, 16), lambda i: (i, 0)),
                  pl.BlockSpec((B, din), lambda i: (i, 0)),
                  pl.BlockSpec((d, din), lambda i: (0, 0)),
                  pl.BlockSpec((1, d), lambda i: (0, 0)),
                  pl.BlockSpec((dn, d), lambda i: (0, 0))],
        out_specs=[pl.BlockSpec((B, d), lambda i: (i, 0)),
                   pl.BlockSpec((B, dhn), lambda i: (i, 0)),
                   pl.BlockSpec((B, dhn), lambda i: (i, 0))],
        out_shape=outs,
    )(s0, s1, c0, c1, h_prev, wr, b.reshape(1, d), wl_next)


def _tc_final(s0, s1, c0, c1, h_prev, wr, b):
    n, din = h_prev.shape
    d = wr.shape[0]
    dh = s0.shape[1]
    B = 2000

    def body(s0_ref, s1_ref, c0_ref, c1_ref, h_ref, wr_ref, b_ref, o_ref):
        s = jnp.concatenate([s0_ref[...], s1_ref[...]],
                            axis=1).astype(jnp.float32)
        cnt = c0_ref[:, 0:1] + c1_ref[:, 0:1]
        inv = 1.0 / jnp.maximum(cnt, 1.0)
        h = s * inv + _dotT(h_ref[...], wr_ref[...]) + b_ref[...]
        h = jnp.maximum(h, 0.0)
        m = jnp.max(h, axis=1, keepdims=True)
        lse = jnp.log(jnp.sum(jnp.exp(h - m), axis=1, keepdims=True))
        o_ref[...] = h - m - lse

    return pl.pallas_call(
        body,
        grid=(n // B,),
        in_specs=[pl.BlockSpec((B, dh), lambda i: (i, 0)),
                  pl.BlockSpec((B, dh), lambda i: (i, 0)),
                  pl.BlockSpec((B, 16), lambda i: (i, 0)),
                  pl.BlockSpec((B, 16), lambda i: (i, 0)),
                  pl.BlockSpec((B, din), lambda i: (i, 0)),
                  pl.BlockSpec((d, din), lambda i: (0, 0)),
                  pl.BlockSpec((1, d), lambda i: (0, 0))],
        out_specs=pl.BlockSpec((B, d), lambda i: (i, 0)),
        out_shape=jax.ShapeDtypeStruct((n, d), jnp.float32),
    )(s0, s1, c0, c1, h_prev, wr, b.reshape(1, d))


def kernel(x, edge_index, Wl1, bl1, Wr1, Wl2, bl2, Wr2, Wl3, bl3, Wr3):
    n = x.shape[0]
    e = edge_index.shape[1]
    src3 = edge_index[0].reshape(NS, (e // NS) // C, C)
    dst3 = edge_index[1].reshape(NS, (e // NS) // C, C)

    y10, y11 = _tc_pre(x, Wl1)
    s10, s11, c0, c1 = _make_sc_agg(n, e, Wl1.shape[0] // 2, True)(
        y10, y11, src3, dst3)
    h1, y20, y21 = _tc_mid(s10, s11, c0, c1, x, Wr1, bl1, Wl2)
    s20, s21 = _make_sc_agg(n, e, Wl2.shape[0] // 2, False)(
        y20, y21, src3, dst3)
    h2, y30, y31 = _tc_mid(s20, s21, c0, c1, h1, Wr2, bl2, Wl3)
    s30, s31 = _make_sc_agg(n, e, Wl3.shape[0] // 2, False)(
        y30, y31, src3, dst3)
    return _tc_final(s30, s31, c0, c1, h2, Wr3, bl3)
